# Initial kernel scaffold; baseline (speedup 1.0000x reference)
import jax
import jax.numpy as jnp
from jax.experimental import pallas as pl


def kernel(lemma_flat, cu_seqlens, table, Wd, bd, W1, b1, W2, b2, W3, b3):
    raise NotImplementedError("write your pallas kernel here")


# speedup vs baseline: 2.9004x; 2.9004x over previous
import functools

import jax
import jax.numpy as jnp
from jax import lax
from jax.experimental import pallas as pl
from jax.experimental.pallas import tpu as pltpu
from jax.experimental.pallas import tpu_sc as plsc

A = 8192
D = 1024
B = 16
RPB = 64
R = B * RPB
TOTAL = 4096
OUT_LEN = 50
STEPS = 49
START_SYM = 1
STOP_SYM = 2
BPG = 4
NC = 2
NS = 16
NW = NC * NS
RPW = R // NW


def _sc_gather(lemma_flat, cu_pad, table):
    mesh = plsc.VectorSubcoreMesh(core_axis_name="c", subcore_axis_name="s")

    @functools.partial(
        pl.kernel,
        out_type=jax.ShapeDtypeStruct((R, D), jnp.float32),
        mesh=mesh,
        scratch_types=[
            pltpu.VMEM((TOTAL,), jnp.int32),
            pltpu.VMEM((RPW,), jnp.int32),
            pltpu.VMEM((RPW, D), jnp.float32),
            pltpu.VMEM((32,), jnp.int32),
            pltpu.SemaphoreType.DMA,
        ],
        compiler_params=pltpu.CompilerParams(needs_layout_passes=False),
    )
    def k(lemma_hbm, cu_hbm, table_hbm, out_hbm, lemma_v, idx_v, rows_v, cu_v, sem):
        wid = lax.axis_index("s") * NC + lax.axis_index("c")
        b = wid // 2
        half = wid % 2
        pltpu.sync_copy(cu_hbm, cu_v)
        pltpu.sync_copy(lemma_hbm, lemma_v)
        bvec = jnp.full((16,), b, jnp.int32)
        start = plsc.load_gather(cu_v, [bvec])
        for chunk in range(RPW // 16):
            io = lax.broadcasted_iota(jnp.int32, (16,), 0)
            p = start + (half * RPW + chunk * 16 + 1) + io
            p = jnp.minimum(p, TOTAL - 1)
            tok = plsc.load_gather(lemma_v, [p])
            idx_v[pl.ds(chunk * 16, 16)] = tok
        pltpu.async_copy(table_hbm.at[idx_v], rows_v, sem).wait()
        pltpu.sync_copy(rows_v, out_hbm.at[pl.ds(wid * RPW, RPW)])

    return k(lemma_flat, cu_pad, table)


def _tc_body(cu_ref, e_ref, wd_ref, bd_ref, o_ref):
    pid = pl.program_id(0)
    m = BPG * RPB
    i = lax.broadcasted_iota(jnp.int32, (m, 1), 0)
    li = i % RPB
    g = i // RPB
    ln = jnp.zeros((m, 1), jnp.int32)
    for kk in range(BPG):
        lnk = cu_ref[pid * BPG + kk + 1] - cu_ref[pid * BPG + kk]
        ln = jnp.where(g == kk, lnk, ln)
    valid = ((li + 1) < ln) & (li < STEPS)
    e = e_ref[0] * valid.astype(jnp.float32)
    x = jnp.dot(e.astype(jnp.bfloat16), wd_ref[...],
                preferred_element_type=jnp.float32)
    x = x + bd_ref[...]
    s = jax.nn.sigmoid(x)
    mx = jnp.max(s, axis=-1, keepdims=True)
    ex = jnp.exp(s - mx)
    denom = jnp.sum(ex, axis=-1, keepdims=True)
    y = ex / denom
    ymax = jnp.max(y, axis=-1, keepdims=True)
    lane = lax.broadcasted_iota(jnp.int32, (m, A), 1)
    sym = jnp.min(jnp.where(y == ymax, lane, A), axis=-1, keepdims=True)
    hit = ((sym == STOP_SYM) & (li < STEPS)).astype(jnp.bfloat16)
    r_i = lax.broadcasted_iota(jnp.int32, (m, m), 0)
    r_k = lax.broadcasted_iota(jnp.int32, (m, m), 1)
    tril = ((r_i // RPB == r_k // RPB) & (r_k < r_i)).astype(jnp.bfloat16)
    wb = jnp.dot(tril, hit, preferred_element_type=jnp.float32)
    keep = wb == 0.0
    stoprow = (lane == STOP_SYM).astype(jnp.float32)
    body = jnp.where(keep, y, stoprow)
    startrow = (lax.broadcasted_iota(jnp.int32, (1, A), 1) == START_SYM)
    for kk in range(BPG):
        o_ref[kk, 0:1, :] = startrow.astype(jnp.float32)
        o_ref[kk, 1:OUT_LEN, :] = body[kk * RPB: kk * RPB + STEPS, :]


def _tc_decode(cu_pad, e_all, wd16, bd_row):
    m = BPG * RPB
    return pl.pallas_call(
        _tc_body,
        grid=(B // BPG,),
        in_specs=[
            pl.BlockSpec(memory_space=pltpu.SMEM),
            pl.BlockSpec((1, m, D), lambda i: (i, 0, 0)),
            pl.BlockSpec((D, A), lambda i: (0, 0)),
            pl.BlockSpec((1, A), lambda i: (0, 0)),
        ],
        out_specs=pl.BlockSpec((BPG, OUT_LEN, A), lambda i: (i, 0, 0)),
        out_shape=jax.ShapeDtypeStruct((B, OUT_LEN, A), jnp.float32),
        compiler_params=pltpu.CompilerParams(
            dimension_semantics=("arbitrary",),
            vmem_limit_bytes=128 * 1024 * 1024,
        ),
    )(cu_pad, e_all, wd16, bd_row)


def kernel(lemma_flat, cu_seqlens, table, Wd, bd, W1, b1, W2, b2, W3, b3):
    cu_pad = jnp.pad(cu_seqlens, (0, 32 - cu_seqlens.shape[0]))
    e_rows = _sc_gather(lemma_flat, cu_pad, table)
    e_all = e_rows.reshape(B // BPG, BPG * RPB, D)
    wd16 = Wd.astype(jnp.bfloat16)
    bd_row = bd.reshape(1, A)
    return _tc_decode(cu_pad, e_all, wd16, bd_row)


# speedup vs baseline: 2.9303x; 1.0103x over previous
import functools

import jax
import jax.numpy as jnp
from jax import lax
from jax.experimental import pallas as pl
from jax.experimental.pallas import tpu as pltpu
from jax.experimental.pallas import tpu_sc as plsc

A = 8192
D = 1024
B = 16
RPB = 64
R = B * RPB
TOTAL = 4096
OUT_LEN = 50
STEPS = 49
START_SYM = 1
STOP_SYM = 2
BPG = 4
NC = 2
NS = 16
NW = NC * NS
RPW = R // NW


def _sc_gather(lemma_flat, cu_pad, table):
    mesh = plsc.VectorSubcoreMesh(core_axis_name="c", subcore_axis_name="s")

    @functools.partial(
        pl.kernel,
        out_type=jax.ShapeDtypeStruct((R, D), jnp.float32),
        mesh=mesh,
        scratch_types=[
            pltpu.VMEM((TOTAL,), jnp.int32),
            pltpu.VMEM((RPW,), jnp.int32),
            pltpu.VMEM((RPW, D), jnp.float32),
            pltpu.VMEM((32,), jnp.int32),
            pltpu.SemaphoreType.DMA,
        ],
        compiler_params=pltpu.CompilerParams(needs_layout_passes=False),
    )
    def k(lemma_hbm, cu_hbm, table_hbm, out_hbm, lemma_v, idx_v, rows_v, cu_v, sem):
        wid = lax.axis_index("s") * NC + lax.axis_index("c")
        b = wid // 2
        half = wid % 2
        pltpu.sync_copy(cu_hbm, cu_v)
        pltpu.sync_copy(lemma_hbm, lemma_v)
        bvec = jnp.full((16,), b, jnp.int32)
        start = plsc.load_gather(cu_v, [bvec])
        for chunk in range(RPW // 16):
            io = lax.broadcasted_iota(jnp.int32, (16,), 0)
            p = start + (half * RPW + chunk * 16 + 1) + io
            p = jnp.minimum(p, TOTAL - 1)
            tok = plsc.load_gather(lemma_v, [p])
            idx_v[pl.ds(chunk * 16, 16)] = tok
        pltpu.async_copy(table_hbm.at[idx_v], rows_v, sem).wait()
        pltpu.sync_copy(rows_v, out_hbm.at[pl.ds(wid * RPW, RPW)])

    return k(lemma_flat, cu_pad, table)


def _tc_body(cu_ref, e_ref, wd_ref, bd_ref, o_ref):
    pid = pl.program_id(0)
    m = BPG * RPB
    i = lax.broadcasted_iota(jnp.int32, (m, 1), 0)
    li = i % RPB
    g = i // RPB
    ln = jnp.zeros((m, 1), jnp.int32)
    for kk in range(BPG):
        lnk = cu_ref[pid * BPG + kk + 1] - cu_ref[pid * BPG + kk]
        ln = jnp.where(g == kk, lnk, ln)
    valid = ((li + 1) < ln) & (li < STEPS)
    e = e_ref[0] * valid.astype(jnp.float32)
    x = jnp.dot(e.astype(jnp.bfloat16), wd_ref[...],
                preferred_element_type=jnp.float32)
    x = x + bd_ref[...]
    s = jax.nn.sigmoid(x)
    ex = jnp.exp(s)
    denom = jnp.sum(ex, axis=-1, keepdims=True)
    y = ex / denom
    xmax = jnp.max(x, axis=-1, keepdims=True)
    lane = lax.broadcasted_iota(jnp.int32, (m, A), 1)
    sym = jnp.min(jnp.where(x == xmax, lane, A), axis=-1, keepdims=True)
    hit = ((sym == STOP_SYM) & (li < STEPS)).astype(jnp.bfloat16)
    r_i = lax.broadcasted_iota(jnp.int32, (m, m), 0)
    r_k = lax.broadcasted_iota(jnp.int32, (m, m), 1)
    tril = ((r_i // RPB == r_k // RPB) & (r_k < r_i)).astype(jnp.bfloat16)
    wb = jnp.dot(tril, hit, preferred_element_type=jnp.float32)
    keep = wb == 0.0
    stoprow = (lane == STOP_SYM).astype(jnp.float32)
    body = jnp.where(keep, y, stoprow)
    startrow = (lax.broadcasted_iota(jnp.int32, (1, A), 1) == START_SYM)
    for kk in range(BPG):
        o_ref[kk, 0:1, :] = startrow.astype(jnp.float32)
        o_ref[kk, 1:OUT_LEN, :] = body[kk * RPB: kk * RPB + STEPS, :]


def _tc_decode(cu_pad, e_all, wd16, bd_row):
    m = BPG * RPB
    return pl.pallas_call(
        _tc_body,
        grid=(B // BPG,),
        in_specs=[
            pl.BlockSpec(memory_space=pltpu.SMEM),
            pl.BlockSpec((1, m, D), lambda i: (i, 0, 0)),
            pl.BlockSpec((D, A), lambda i: (0, 0)),
            pl.BlockSpec((1, A), lambda i: (0, 0)),
        ],
        out_specs=pl.BlockSpec((BPG, OUT_LEN, A), lambda i: (i, 0, 0)),
        out_shape=jax.ShapeDtypeStruct((B, OUT_LEN, A), jnp.float32),
        compiler_params=pltpu.CompilerParams(
            dimension_semantics=("parallel",),
            vmem_limit_bytes=128 * 1024 * 1024,
        ),
    )(cu_pad, e_all, wd16, bd_row)


def kernel(lemma_flat, cu_seqlens, table, Wd, bd, W1, b1, W2, b2, W3, b3):
    cu_pad = jnp.pad(cu_seqlens, (0, 32 - cu_seqlens.shape[0]))
    e_rows = _sc_gather(lemma_flat, cu_pad, table)
    e_all = e_rows.reshape(B // BPG, BPG * RPB, D)
    wd16 = Wd.astype(jnp.bfloat16)
    bd_row = bd.reshape(1, A)
    return _tc_decode(cu_pad, e_all, wd16, bd_row)


# speedup vs baseline: 3.1933x; 1.0898x over previous
import functools

import jax
import jax.numpy as jnp
from jax import lax
from jax.experimental import pallas as pl
from jax.experimental.pallas import tpu as pltpu
from jax.experimental.pallas import tpu_sc as plsc

A = 8192
D = 1024
B = 16
RPB = 64
R = B * RPB
TOTAL = 4096
OUT_LEN = 50
STEPS = 49
START_SYM = 1
STOP_SYM = 2
BPG = 4
NC = 2
NS = 16
NW = NC * NS
RPW = R // NW


def _sc_gather(cu_lemma, table):
    mesh = plsc.VectorSubcoreMesh(core_axis_name="c", subcore_axis_name="s")

    @functools.partial(
        pl.kernel,
        out_type=jax.ShapeDtypeStruct((R, D), jnp.float32),
        mesh=mesh,
        scratch_types=[
            pltpu.VMEM((32 + TOTAL,), jnp.int32),
            pltpu.VMEM((RPW,), jnp.int32),
            pltpu.VMEM((RPW, D), jnp.float32),
            pltpu.SemaphoreType.DMA,
        ],
        compiler_params=pltpu.CompilerParams(needs_layout_passes=False),
    )
    def k(cl_hbm, table_hbm, out_hbm, cl_v, idx_v, rows_v, sem):
        wid = lax.axis_index("s") * NC + lax.axis_index("c")
        b = wid // 2
        half = wid % 2
        pltpu.sync_copy(cl_hbm, cl_v)
        bvec = jnp.full((16,), b, jnp.int32)
        start = plsc.load_gather(cl_v, [bvec])
        for chunk in range(RPW // 16):
            io = lax.broadcasted_iota(jnp.int32, (16,), 0)
            p = start + (half * RPW + chunk * 16 + 1) + io
            p = jnp.minimum(p, TOTAL - 1) + 32
            tok = plsc.load_gather(cl_v, [p])
            idx_v[pl.ds(chunk * 16, 16)] = tok
        pltpu.async_copy(table_hbm.at[idx_v], rows_v, sem).wait()
        pltpu.sync_copy(rows_v, out_hbm.at[pl.ds(wid * RPW, RPW)])

    return k(cu_lemma, table)


NT = 8
TW = A // NT


def _tc_body(cu_ref, e_ref, wd_hbm, bd_ref, o_ref, wd16_ref, stage_ref, sem):
    pid = pl.program_id(0)
    m = BPG * RPB

    @pl.when(pid == 0)
    def _load_wd():
        cps = [
            pltpu.make_async_copy(
                wd_hbm.at[:, pl.ds(t * TW, TW)], stage_ref.at[t % 2], sem.at[t % 2]
            )
            for t in range(NT)
        ]
        cps[0].start()
        for t in range(NT):
            if t + 1 < NT:
                cps[t + 1].start()
            cps[t].wait()
            wd16_ref[:, pl.ds(t * TW, TW)] = stage_ref[t % 2].astype(jnp.bfloat16)
    i = lax.broadcasted_iota(jnp.int32, (m, 1), 0)
    li = i % RPB
    g = i // RPB
    ln = jnp.zeros((m, 1), jnp.int32)
    for kk in range(BPG):
        lnk = cu_ref[pid * BPG + kk + 1] - cu_ref[pid * BPG + kk]
        ln = jnp.where(g == kk, lnk, ln)
    valid = ((li + 1) < ln) & (li < STEPS)
    e = e_ref[0] * valid.astype(jnp.float32)
    x = jnp.dot(e.astype(jnp.bfloat16), wd16_ref[...],
                preferred_element_type=jnp.float32)
    x = x + bd_ref[...]
    s = jax.nn.sigmoid(x)
    ex = jnp.exp(s)
    denom = jnp.sum(ex, axis=-1, keepdims=True)
    y = ex / denom
    xmax = jnp.max(x, axis=-1, keepdims=True)
    lane = lax.broadcasted_iota(jnp.int32, (m, A), 1)
    sym = jnp.min(jnp.where(x == xmax, lane, A), axis=-1, keepdims=True)
    hit = ((sym == STOP_SYM) & (li < STEPS)).astype(jnp.bfloat16)
    r_i = lax.broadcasted_iota(jnp.int32, (m, m), 0)
    r_k = lax.broadcasted_iota(jnp.int32, (m, m), 1)
    tril = ((r_i // RPB == r_k // RPB) & (r_k < r_i)).astype(jnp.bfloat16)
    wb = jnp.dot(tril, hit, preferred_element_type=jnp.float32)
    keep = wb == 0.0
    stoprow = (lane == STOP_SYM).astype(jnp.float32)
    body = jnp.where(keep, y, stoprow)
    startrow = (lax.broadcasted_iota(jnp.int32, (1, A), 1) == START_SYM)
    for kk in range(BPG):
        o_ref[kk, 0:1, :] = startrow.astype(jnp.float32)
        o_ref[kk, 1:OUT_LEN, :] = body[kk * RPB: kk * RPB + STEPS, :]


def _tc_decode(cu_pad, e_all, wd, bd_row):
    m = BPG * RPB
    return pl.pallas_call(
        _tc_body,
        grid=(B // BPG,),
        in_specs=[
            pl.BlockSpec(memory_space=pltpu.SMEM),
            pl.BlockSpec((1, m, D), lambda i: (i, 0, 0)),
            pl.BlockSpec(memory_space=pl.ANY),
            pl.BlockSpec((1, A), lambda i: (0, 0)),
        ],
        out_specs=pl.BlockSpec((BPG, OUT_LEN, A), lambda i: (i, 0, 0)),
        out_shape=jax.ShapeDtypeStruct((B, OUT_LEN, A), jnp.float32),
        scratch_shapes=[
            pltpu.VMEM((D, A), jnp.bfloat16),
            pltpu.VMEM((2, D, TW), jnp.float32),
            pltpu.SemaphoreType.DMA((2,)),
        ],
        compiler_params=pltpu.CompilerParams(
            dimension_semantics=("arbitrary",),
            vmem_limit_bytes=128 * 1024 * 1024,
        ),
    )(cu_pad, e_all, wd, bd_row)


def kernel(lemma_flat, cu_seqlens, table, Wd, bd, W1, b1, W2, b2, W3, b3):
    cu_pad = jnp.pad(cu_seqlens, (0, 32 - cu_seqlens.shape[0]))
    cu_lemma = jnp.concatenate([cu_pad, lemma_flat])
    e_rows = _sc_gather(cu_lemma, table)
    e_all = e_rows.reshape(B // BPG, BPG * RPB, D)
    bd_row = bd.reshape(1, A)
    return _tc_decode(cu_pad, e_all, Wd, bd_row)
